# Initial kernel scaffold; baseline (speedup 1.0000x reference)
import jax
import jax.numpy as jnp
from jax.experimental import pallas as pl


def kernel(leak_area, edge_index, pipe_attrs, params):
    raise NotImplementedError("write your pallas kernel here")


# speedup vs baseline: 3.4042x; 3.4042x over previous
import functools

import jax
import jax.numpy as jnp
from jax import lax
from jax.experimental import pallas as pl
from jax.experimental.pallas import tpu as pltpu
from jax.experimental.pallas import tpu_sc as plsc

N = 50000
E = 1600000
D = 16
ATTR = 4

NC = 2
NS = 16
NW = NC * NS

CH = 80
NCH = 25
SB = CH * NCH
EPW = E // NW
NSB = EPW // SB
ROWS_PW = EPW // CH
NZ = N // SB


def _mesh():
    return plsc.VectorSubcoreMesh(core_axis_name="c", subcore_axis_name="s")


def _sc_gather(x, src, dst):
    @functools.partial(
        pl.kernel,
        out_type=(jax.ShapeDtypeStruct((E, D), jnp.float32),
                  jax.ShapeDtypeStruct((E, D), jnp.float32)),
        mesh=_mesh(),
        compiler_params=pltpu.CompilerParams(use_tc_tiling_on_sc=False),
        scratch_types=[
            pltpu.VMEM((SB,), jnp.int32),
            pltpu.VMEM((SB,), jnp.int32),
            pltpu.VMEM((SB, D), jnp.float32),
            pltpu.VMEM((SB, D), jnp.float32),
            pltpu.VMEM_SHARED((N, D), jnp.float32),
            pltpu.SemaphoreType.DMA,
        ],
    )
    def k(x_hbm, src_hbm, dst_hbm, xi_hbm, xj_hbm, idx_s, idx_d, buf_i, buf_j,
          xs, sem):
        cid = lax.axis_index("c")
        sid = lax.axis_index("s")
        wid = sid * NC + cid
        e0 = wid * EPW

        for r in range(2):
            rng = sid + r * NS

            @pl.when(rng < NZ)
            def _():
                pltpu.sync_copy(x_hbm.at[pl.ds(rng * SB, SB)], buf_i)
                pltpu.sync_copy(buf_i, xs.at[pl.ds(rng * SB, SB)])

        plsc.subcore_barrier()

        def sb_body(sb, carry):
            base = e0 + sb * SB
            pltpu.sync_copy(dst_hbm.at[pl.ds(base, SB)], idx_d)
            pltpu.sync_copy(src_hbm.at[pl.ds(base, SB)], idx_s)

            def fire(c, cc):
                pltpu.async_copy(xs.at[idx_d.at[pl.ds(c * CH, CH)]],
                                 buf_i.at[pl.ds(c * CH, CH)], sem)
                pltpu.async_copy(xs.at[idx_s.at[pl.ds(c * CH, CH)]],
                                 buf_j.at[pl.ds(c * CH, CH)], sem)
                return cc

            lax.fori_loop(0, NCH, fire, 0)

            def drain(c, cc):
                pltpu.make_async_copy(
                    xs.at[idx_d.at[pl.ds(c * CH, CH)]],
                    buf_i.at[pl.ds(c * CH, CH)], sem).wait()
                pltpu.make_async_copy(
                    xs.at[idx_s.at[pl.ds(c * CH, CH)]],
                    buf_j.at[pl.ds(c * CH, CH)], sem).wait()
                return cc

            lax.fori_loop(0, NCH, drain, 0)

            pltpu.sync_copy(buf_i, xi_hbm.at[pl.ds(base, SB)])
            pltpu.sync_copy(buf_j, xj_hbm.at[pl.ds(base, SB)])
            return carry

        lax.fori_loop(0, NSB, sb_body, 0)

    return k(x, src, dst)


def _sc_scatter(msg, dst, zeros_sb):
    @functools.partial(
        pl.kernel,
        out_type=jax.ShapeDtypeStruct((NC, N, D), jnp.float32),
        mesh=_mesh(),
        compiler_params=pltpu.CompilerParams(use_tc_tiling_on_sc=False),
        scratch_types=[
            pltpu.VMEM((SB,), jnp.int32),
            pltpu.VMEM((SB, D), jnp.float32),
            pltpu.VMEM_SHARED((N, D), jnp.float32),
        ],
    )
    def k(msg_hbm, dst_hbm, z_hbm, out_hbm, idx_d, buf, acc):
        cid = lax.axis_index("c")
        sid = lax.axis_index("s")
        wid = sid * NC + cid

        pltpu.sync_copy(z_hbm, buf)
        for r in range(2):
            rng = sid + r * NS

            @pl.when(rng < NZ)
            def _():
                pltpu.sync_copy(buf, acc.at[pl.ds(rng * SB, SB)])

        plsc.subcore_barrier()

        e0 = wid * EPW

        def sb_body(sb, carry):
            base = e0 + sb * SB
            pltpu.sync_copy(dst_hbm.at[pl.ds(base, SB)], idx_d)
            pltpu.sync_copy(msg_hbm.at[pl.ds(base, SB)], buf)

            def sc_body(c, cc):
                pltpu.sync_copy(buf.at[pl.ds(c * CH, CH)],
                                acc.at[idx_d.at[pl.ds(c * CH, CH)]], add=True)
                return cc

            lax.fori_loop(0, NCH, sc_body, 0)
            return carry

        lax.fori_loop(0, NSB, sb_body, 0)
        plsc.subcore_barrier()

        for r in range(2):
            rng = sid + r * NS

            @pl.when(rng < NZ)
            def _():
                pltpu.sync_copy(acc.at[pl.ds(rng * SB, SB)], buf)
                pltpu.sync_copy(buf, out_hbm.at[cid, pl.ds(rng * SB, SB)])

    return k(msg, dst, zeros_sb)


def _sc_count(dst, zeros_sb, ones_ch):
    @functools.partial(
        pl.kernel,
        out_type=jax.ShapeDtypeStruct((NC, N, D), jnp.float32),
        mesh=_mesh(),
        compiler_params=pltpu.CompilerParams(use_tc_tiling_on_sc=False),
        scratch_types=[
            pltpu.VMEM((SB,), jnp.int32),
            pltpu.VMEM((SB, D), jnp.float32),
            pltpu.VMEM((CH, D), jnp.float32),
            pltpu.VMEM_SHARED((N, D), jnp.float32),
        ],
    )
    def k(dst_hbm, z_hbm, o_hbm, out_hbm, idx_d, buf, ones, acc):
        cid = lax.axis_index("c")
        sid = lax.axis_index("s")
        wid = sid * NC + cid

        pltpu.sync_copy(z_hbm, buf)
        pltpu.sync_copy(o_hbm, ones)
        for r in range(2):
            rng = sid + r * NS

            @pl.when(rng < NZ)
            def _():
                pltpu.sync_copy(buf, acc.at[pl.ds(rng * SB, SB)])

        plsc.subcore_barrier()

        e0 = wid * EPW

        def sb_body(sb, carry):
            pltpu.sync_copy(dst_hbm.at[pl.ds(e0 + sb * SB, SB)], idx_d)

            def sc_body(c, cc):
                pltpu.sync_copy(ones, acc.at[idx_d.at[pl.ds(c * CH, CH)]],
                                add=True)
                return cc

            lax.fori_loop(0, NCH, sc_body, 0)
            return carry

        lax.fori_loop(0, NSB, sb_body, 0)
        plsc.subcore_barrier()

        for r in range(2):
            rng = sid + r * NS

            @pl.when(rng < NZ)
            def _():
                pltpu.sync_copy(acc.at[pl.ds(rng * SB, SB)], buf)
                pltpu.sync_copy(buf, out_hbm.at[cid, pl.ds(rng * SB, SB)])

    return k(dst, zeros_sb, ones_ch)


def _tc_node_embed(leak, w, b):
    BLK = 10000

    def body(l_ref, w_ref, b_ref, x_ref):
        x_ref[...] = l_ref[...] * w_ref[...] + b_ref[...]

    return pl.pallas_call(
        body,
        grid=(N // BLK,),
        in_specs=[
            pl.BlockSpec((BLK, 1), lambda i: (i, 0)),
            pl.BlockSpec((1, D), lambda i: (0, 0)),
            pl.BlockSpec((1, D), lambda i: (0, 0)),
        ],
        out_specs=pl.BlockSpec((BLK, D), lambda i: (i, 0)),
        out_shape=jax.ShapeDtypeStruct((N, D), jnp.float32),
    )(leak, w, b.reshape(1, D))


def _tc_edge_embed(pipe_attrs, we, be, wq, bq):
    BLK = 6400

    def body(a_ref, we_ref, be_ref, wq_ref, bq_ref, e_ref, q_ref):
        e = jnp.dot(a_ref[...], we_ref[...],
                    preferred_element_type=jnp.float32) + be_ref[...]
        e_ref[...] = e
        q_ref[...] = jnp.sum(e * wq_ref[...], axis=1, keepdims=True) + bq_ref[0, 0]

    return pl.pallas_call(
        body,
        grid=(E // BLK,),
        in_specs=[
            pl.BlockSpec((BLK, ATTR), lambda i: (i, 0)),
            pl.BlockSpec((ATTR, D), lambda i: (0, 0)),
            pl.BlockSpec((1, D), lambda i: (0, 0)),
            pl.BlockSpec((1, D), lambda i: (0, 0)),
            pl.BlockSpec((1, 1), lambda i: (0, 0)),
        ],
        out_specs=[
            pl.BlockSpec((BLK, D), lambda i: (i, 0)),
            pl.BlockSpec((BLK, 1), lambda i: (i, 0)),
        ],
        out_shape=[jax.ShapeDtypeStruct((E, D), jnp.float32),
                   jax.ShapeDtypeStruct((E, 1), jnp.float32)],
    )(pipe_attrs, we, be.reshape(1, D), wq.reshape(1, D), bq.reshape(1, 1))


def _tc_edge_mlp(xi, xj, e, w1, b1, w2, b2):
    BLK = 6400

    def body(xi_ref, xj_ref, e_ref, w1_ref, b1_ref, w2_ref, b2_ref, o_ref):
        m = jnp.concatenate([xi_ref[...], xj_ref[...], e_ref[...]], axis=1)
        h = jnp.maximum(
            jnp.dot(m, w1_ref[...], preferred_element_type=jnp.float32)
            + b1_ref[...], 0.0)
        o_ref[...] = jnp.dot(h, w2_ref[...],
                             preferred_element_type=jnp.float32) + b2_ref[...]

    return pl.pallas_call(
        body,
        grid=(E // BLK,),
        in_specs=[
            pl.BlockSpec((BLK, D), lambda i: (i, 0)),
            pl.BlockSpec((BLK, D), lambda i: (i, 0)),
            pl.BlockSpec((BLK, D), lambda i: (i, 0)),
            pl.BlockSpec((3 * D, 4 * D), lambda i: (0, 0)),
            pl.BlockSpec((1, 4 * D), lambda i: (0, 0)),
            pl.BlockSpec((4 * D, D), lambda i: (0, 0)),
            pl.BlockSpec((1, D), lambda i: (0, 0)),
        ],
        out_specs=pl.BlockSpec((BLK, D), lambda i: (i, 0)),
        out_shape=jax.ShapeDtypeStruct((E, D), jnp.float32),
    )(xi, xj, e, w1, b1.reshape(1, 4 * D), w2, b2.reshape(1, D))


def _tc_node_mlp(x, p0, p1, c0, c1, w1, b1, w2, b2):
    BLK = 5000

    def body(x_ref, p0_ref, p1_ref, c0_ref, c1_ref, w1_ref, b1_ref, w2_ref,
             b2_ref, o_ref):
        aggr = (p0_ref[...] + p1_ref[...]) / jnp.maximum(
            c0_ref[...] + c1_ref[...], 1.0)
        u = jnp.concatenate([x_ref[...], aggr], axis=1)
        h = jnp.maximum(
            jnp.dot(u, w1_ref[...], preferred_element_type=jnp.float32)
            + b1_ref[...], 0.0)
        o_ref[...] = jnp.dot(h, w2_ref[...],
                             preferred_element_type=jnp.float32) + b2_ref[...]

    return pl.pallas_call(
        body,
        grid=(N // BLK,),
        in_specs=[pl.BlockSpec((BLK, D), lambda i: (i, 0))] * 5 + [
            pl.BlockSpec((2 * D, 4 * D), lambda i: (0, 0)),
            pl.BlockSpec((1, 4 * D), lambda i: (0, 0)),
            pl.BlockSpec((4 * D, D), lambda i: (0, 0)),
            pl.BlockSpec((1, D), lambda i: (0, 0)),
        ],
        out_specs=pl.BlockSpec((BLK, D), lambda i: (i, 0)),
        out_shape=jax.ShapeDtypeStruct((N, D), jnp.float32),
    )(x, p0, p1, c0, c1, w1, b1.reshape(1, 4 * D), w2, b2.reshape(1, D))


def _tc_final(x, w, b):
    BLK = 10000

    def body(x_ref, w_ref, b_ref, h_ref):
        h_ref[...] = jnp.sum(x_ref[...] * w_ref[...], axis=1,
                             keepdims=True) + b_ref[0, 0]

    return pl.pallas_call(
        body,
        grid=(N // BLK,),
        in_specs=[
            pl.BlockSpec((BLK, D), lambda i: (i, 0)),
            pl.BlockSpec((1, D), lambda i: (0, 0)),
            pl.BlockSpec((1, 1), lambda i: (0, 0)),
        ],
        out_specs=pl.BlockSpec((BLK, 1), lambda i: (i, 0)),
        out_shape=jax.ShapeDtypeStruct((N, 1), jnp.float32),
    )(x, w.reshape(1, D), b.reshape(1, 1))


def kernel(leak_area, edge_index, pipe_attrs, params):
    src = edge_index[0]
    dst = edge_index[1]
    zeros_sb = jnp.zeros((SB, D), jnp.float32)
    ones_ch = jnp.ones((CH, D), jnp.float32)

    x = _tc_node_embed(leak_area, params['node_embed'][0], params['node_embed'][1])
    e, q = _tc_edge_embed(pipe_attrs, params['edge_embed'][0],
                          params['edge_embed'][1], params['final_edge'][0],
                          params['final_edge'][1])
    cnt = _sc_count(dst, zeros_sb, ones_ch)

    for lp in params['layers']:
        xi, xj = _sc_gather(x, src, dst)
        msg = _tc_edge_mlp(xi, xj, e, lp['e1'][0], lp['e1'][1],
                           lp['e2'][0], lp['e2'][1])
        parts = _sc_scatter(msg, dst, zeros_sb)
        x = _tc_node_mlp(x, parts[0], parts[1], cnt[0], cnt[1],
                         lp['n1'][0], lp['n1'][1], lp['n2'][0], lp['n2'][1])

    H = _tc_final(x, params['final_node'][0], params['final_node'][1])
    return (H.reshape(N), q.reshape(E))


# speedup vs baseline: 8.0690x; 2.3703x over previous
import functools

import jax
import jax.numpy as jnp
from jax import lax
from jax.experimental import pallas as pl
from jax.experimental.pallas import tpu as pltpu
from jax.experimental.pallas import tpu_sc as plsc

N = 50000
E = 1600000
D = 16
ATTR = 4

NC = 2
NS = 16
NW = NC * NS

CH = 80
NCH = 25
SB = CH * NCH
EPW = E // NW
NSB = EPW // SB
NZ = N // SB

E8 = E // 8
N8 = N // 8
E32 = E // 32


def _mesh():
    return plsc.VectorSubcoreMesh(core_axis_name="c", subcore_axis_name="s")


def _bd(w, k):
    return jnp.kron(jnp.eye(k, dtype=w.dtype), w)


def _sc_gather(x, src, dst):
    @functools.partial(
        pl.kernel,
        out_type=(jax.ShapeDtypeStruct((E, D), jnp.float32),
                  jax.ShapeDtypeStruct((E, D), jnp.float32)),
        mesh=_mesh(),
        compiler_params=pltpu.CompilerParams(use_tc_tiling_on_sc=False),
        scratch_types=[
            pltpu.VMEM((SB,), jnp.int32),
            pltpu.VMEM((SB,), jnp.int32),
            pltpu.VMEM((SB, D), jnp.float32),
            pltpu.VMEM((SB, D), jnp.float32),
            pltpu.VMEM_SHARED((N, D), jnp.float32),
            pltpu.SemaphoreType.DMA,
        ],
    )
    def k(x_hbm, src_hbm, dst_hbm, xi_hbm, xj_hbm, idx_s, idx_d, buf_i, buf_j,
          xs, sem):
        cid = lax.axis_index("c")
        sid = lax.axis_index("s")
        wid = sid * NC + cid
        e0 = wid * EPW

        for r in range(2):
            rng = sid + r * NS

            @pl.when(rng < NZ)
            def _():
                pltpu.sync_copy(x_hbm.at[pl.ds(rng * SB, SB)], buf_i)
                pltpu.sync_copy(buf_i, xs.at[pl.ds(rng * SB, SB)])

        plsc.subcore_barrier()

        def sb_body(sb, carry):
            base = e0 + sb * SB
            pltpu.sync_copy(dst_hbm.at[pl.ds(base, SB)], idx_d)
            pltpu.sync_copy(src_hbm.at[pl.ds(base, SB)], idx_s)

            def fire(c, cc):
                pltpu.async_copy(xs.at[idx_d.at[pl.ds(c * CH, CH)]],
                                 buf_i.at[pl.ds(c * CH, CH)], sem)
                pltpu.async_copy(xs.at[idx_s.at[pl.ds(c * CH, CH)]],
                                 buf_j.at[pl.ds(c * CH, CH)], sem)
                return cc

            lax.fori_loop(0, NCH, fire, 0)

            def drain(c, cc):
                pltpu.make_async_copy(
                    xs.at[idx_d.at[pl.ds(c * CH, CH)]],
                    buf_i.at[pl.ds(c * CH, CH)], sem).wait()
                pltpu.make_async_copy(
                    xs.at[idx_s.at[pl.ds(c * CH, CH)]],
                    buf_j.at[pl.ds(c * CH, CH)], sem).wait()
                return cc

            lax.fori_loop(0, NCH, drain, 0)

            pltpu.sync_copy(buf_i, xi_hbm.at[pl.ds(base, SB)])
            pltpu.sync_copy(buf_j, xj_hbm.at[pl.ds(base, SB)])
            return carry

        lax.fori_loop(0, NSB, sb_body, 0)

    return k(x, src, dst)


def _sc_scatter(msg, dst, zeros_sb):
    @functools.partial(
        pl.kernel,
        out_type=jax.ShapeDtypeStruct((NC, N, D), jnp.float32),
        mesh=_mesh(),
        compiler_params=pltpu.CompilerParams(use_tc_tiling_on_sc=False),
        scratch_types=[
            pltpu.VMEM((SB,), jnp.int32),
            pltpu.VMEM((SB, D), jnp.float32),
            pltpu.VMEM_SHARED((N, D), jnp.float32),
        ],
    )
    def k(msg_hbm, dst_hbm, z_hbm, out_hbm, idx_d, buf, acc):
        cid = lax.axis_index("c")
        sid = lax.axis_index("s")
        wid = sid * NC + cid

        pltpu.sync_copy(z_hbm, buf)
        for r in range(2):
            rng = sid + r * NS

            @pl.when(rng < NZ)
            def _():
                pltpu.sync_copy(buf, acc.at[pl.ds(rng * SB, SB)])

        plsc.subcore_barrier()

        e0 = wid * EPW

        def sb_body(sb, carry):
            base = e0 + sb * SB
            pltpu.sync_copy(dst_hbm.at[pl.ds(base, SB)], idx_d)
            pltpu.sync_copy(msg_hbm.at[pl.ds(base, SB)], buf)

            def sc_body(c, cc):
                pltpu.sync_copy(buf.at[pl.ds(c * CH, CH)],
                                acc.at[idx_d.at[pl.ds(c * CH, CH)]], add=True)
                return cc

            lax.fori_loop(0, NCH, sc_body, 0)
            return carry

        lax.fori_loop(0, NSB, sb_body, 0)
        plsc.subcore_barrier()

        for r in range(2):
            rng = sid + r * NS

            @pl.when(rng < NZ)
            def _():
                pltpu.sync_copy(acc.at[pl.ds(rng * SB, SB)], buf)
                pltpu.sync_copy(buf, out_hbm.at[cid, pl.ds(rng * SB, SB)])

    return k(msg, dst, zeros_sb)


def _sc_count(dst, zeros_sb, ones_ch):
    @functools.partial(
        pl.kernel,
        out_type=jax.ShapeDtypeStruct((NC, N, D), jnp.float32),
        mesh=_mesh(),
        compiler_params=pltpu.CompilerParams(use_tc_tiling_on_sc=False),
        scratch_types=[
            pltpu.VMEM((SB,), jnp.int32),
            pltpu.VMEM((SB, D), jnp.float32),
            pltpu.VMEM((CH, D), jnp.float32),
            pltpu.VMEM_SHARED((N, D), jnp.float32),
        ],
    )
    def k(dst_hbm, z_hbm, o_hbm, out_hbm, idx_d, buf, ones, acc):
        cid = lax.axis_index("c")
        sid = lax.axis_index("s")
        wid = sid * NC + cid

        pltpu.sync_copy(z_hbm, buf)
        pltpu.sync_copy(o_hbm, ones)
        for r in range(2):
            rng = sid + r * NS

            @pl.when(rng < NZ)
            def _():
                pltpu.sync_copy(buf, acc.at[pl.ds(rng * SB, SB)])

        plsc.subcore_barrier()

        e0 = wid * EPW

        def sb_body(sb, carry):
            pltpu.sync_copy(dst_hbm.at[pl.ds(e0 + sb * SB, SB)], idx_d)

            def sc_body(c, cc):
                pltpu.sync_copy(ones, acc.at[idx_d.at[pl.ds(c * CH, CH)]],
                                add=True)
                return cc

            lax.fori_loop(0, NCH, sc_body, 0)
            return carry

        lax.fori_loop(0, NSB, sb_body, 0)
        plsc.subcore_barrier()

        for r in range(2):
            rng = sid + r * NS

            @pl.when(rng < NZ)
            def _():
                pltpu.sync_copy(acc.at[pl.ds(rng * SB, SB)], buf)
                pltpu.sync_copy(buf, out_hbm.at[cid, pl.ds(rng * SB, SB)])

    return k(dst, zeros_sb, ones_ch)


def _tc_node_embed(leak8, w, b):
    s = jnp.kron(jnp.eye(8, dtype=jnp.float32), w.reshape(1, D))
    bt = jnp.tile(b, 8).reshape(1, 8 * D)

    def body(l_ref, s_ref, b_ref, x_ref):
        x_ref[...] = jnp.dot(l_ref[...], s_ref[...],
                             preferred_element_type=jnp.float32) + b_ref[...]

    return pl.pallas_call(
        body,
        grid=(1,),
        in_specs=[
            pl.BlockSpec((N8, 8), lambda i: (0, 0)),
            pl.BlockSpec((8, 8 * D), lambda i: (0, 0)),
            pl.BlockSpec((1, 8 * D), lambda i: (0, 0)),
        ],
        out_specs=pl.BlockSpec((N8, 8 * D), lambda i: (0, 0)),
        out_shape=jax.ShapeDtypeStruct((N8, 8 * D), jnp.float32),
    )(leak8, s, bt)


def _tc_edge_embed(pa32, we, be, wq, bq):
    bwe = _bd(we, 32)
    bet = jnp.tile(be, 32).reshape(1, 32 * D)
    bwq = _bd(wq.reshape(D, 1), 32)
    BLK = 5000

    def body(a_ref, we_ref, be_ref, wq_ref, bq_ref, e_ref, q_ref):
        e = jnp.dot(a_ref[...], we_ref[...],
                    preferred_element_type=jnp.float32) + be_ref[...]
        e_ref[...] = e
        q_ref[...] = jnp.dot(e, wq_ref[...],
                             preferred_element_type=jnp.float32) + bq_ref[0, 0]

    return pl.pallas_call(
        body,
        grid=(E32 // BLK,),
        in_specs=[
            pl.BlockSpec((BLK, 128), lambda i: (i, 0)),
            pl.BlockSpec((128, 32 * D), lambda i: (0, 0)),
            pl.BlockSpec((1, 32 * D), lambda i: (0, 0)),
            pl.BlockSpec((32 * D, 32), lambda i: (0, 0)),
            pl.BlockSpec((1, 1), lambda i: (0, 0)),
        ],
        out_specs=[
            pl.BlockSpec((BLK, 32 * D), lambda i: (i, 0)),
            pl.BlockSpec((BLK, 32), lambda i: (i, 0)),
        ],
        out_shape=[jax.ShapeDtypeStruct((E32, 32 * D), jnp.float32),
                   jax.ShapeDtypeStruct((E32, 32), jnp.float32)],
    )(pa32, bwe, bet, bwq, bq.reshape(1, 1))


def _tc_edge_mlp8(xi8, xj8, e8, w1, b1, w2, b2):
    bwi = _bd(w1[0:D], 8)
    bwj = _bd(w1[D:2 * D], 8)
    bwe = _bd(w1[2 * D:3 * D], 8)
    b1t = jnp.tile(b1, 8).reshape(1, 512)
    bw2 = _bd(w2, 8)
    b2t = jnp.tile(b2, 8).reshape(1, 128)
    BLK = 2000

    def body(xi_ref, xj_ref, e_ref, wi_ref, wj_ref, we_ref, b1_ref, w2_ref,
             b2_ref, o_ref):
        h = jnp.maximum(
            jnp.dot(xi_ref[...], wi_ref[...], preferred_element_type=jnp.float32)
            + jnp.dot(xj_ref[...], wj_ref[...], preferred_element_type=jnp.float32)
            + jnp.dot(e_ref[...], we_ref[...], preferred_element_type=jnp.float32)
            + b1_ref[...], 0.0)
        o_ref[...] = jnp.dot(h, w2_ref[...],
                             preferred_element_type=jnp.float32) + b2_ref[...]

    return pl.pallas_call(
        body,
        grid=(E8 // BLK,),
        in_specs=[
            pl.BlockSpec((BLK, 128), lambda i: (i, 0)),
            pl.BlockSpec((BLK, 128), lambda i: (i, 0)),
            pl.BlockSpec((BLK, 128), lambda i: (i, 0)),
            pl.BlockSpec((128, 512), lambda i: (0, 0)),
            pl.BlockSpec((128, 512), lambda i: (0, 0)),
            pl.BlockSpec((128, 512), lambda i: (0, 0)),
            pl.BlockSpec((1, 512), lambda i: (0, 0)),
            pl.BlockSpec((512, 128), lambda i: (0, 0)),
            pl.BlockSpec((1, 128), lambda i: (0, 0)),
        ],
        out_specs=pl.BlockSpec((BLK, 128), lambda i: (i, 0)),
        out_shape=jax.ShapeDtypeStruct((E8, 128), jnp.float32),
    )(xi8, xj8, e8, bwi, bwj, bwe, b1t, bw2, b2t)


def _tc_node_mlp8(x8, p0, p1, c0, c1, w1, b1, w2, b2):
    bwx = _bd(w1[0:D], 8)
    bwa = _bd(w1[D:2 * D], 8)
    b1t = jnp.tile(b1, 8).reshape(1, 512)
    bw2 = _bd(w2, 8)
    b2t = jnp.tile(b2, 8).reshape(1, 128)

    def body(x_ref, p0_ref, p1_ref, c0_ref, c1_ref, wx_ref, wa_ref, b1_ref,
             w2_ref, b2_ref, o_ref):
        aggr = (p0_ref[...] + p1_ref[...]) / jnp.maximum(
            c0_ref[...] + c1_ref[...], 1.0)
        h = jnp.maximum(
            jnp.dot(x_ref[...], wx_ref[...], preferred_element_type=jnp.float32)
            + jnp.dot(aggr, wa_ref[...], preferred_element_type=jnp.float32)
            + b1_ref[...], 0.0)
        o_ref[...] = jnp.dot(h, w2_ref[...],
                             preferred_element_type=jnp.float32) + b2_ref[...]

    return pl.pallas_call(
        body,
        grid=(1,),
        in_specs=[pl.BlockSpec((N8, 128), lambda i: (0, 0))] * 5 + [
            pl.BlockSpec((128, 512), lambda i: (0, 0)),
            pl.BlockSpec((128, 512), lambda i: (0, 0)),
            pl.BlockSpec((1, 512), lambda i: (0, 0)),
            pl.BlockSpec((512, 128), lambda i: (0, 0)),
            pl.BlockSpec((1, 128), lambda i: (0, 0)),
        ],
        out_specs=pl.BlockSpec((N8, 128), lambda i: (0, 0)),
        out_shape=jax.ShapeDtypeStruct((N8, 128), jnp.float32),
    )(x8, p0, p1, c0, c1, bwx, bwa, b1t, bw2, b2t)


def _tc_final(x8, w, b):
    bwf = _bd(w.reshape(D, 1), 8)

    def body(x_ref, w_ref, b_ref, h_ref):
        h_ref[...] = jnp.dot(x_ref[...], w_ref[...],
                             preferred_element_type=jnp.float32) + b_ref[0, 0]

    return pl.pallas_call(
        body,
        grid=(1,),
        in_specs=[
            pl.BlockSpec((N8, 128), lambda i: (0, 0)),
            pl.BlockSpec((128, 8), lambda i: (0, 0)),
            pl.BlockSpec((1, 1), lambda i: (0, 0)),
        ],
        out_specs=pl.BlockSpec((N8, 8), lambda i: (0, 0)),
        out_shape=jax.ShapeDtypeStruct((N8, 8), jnp.float32),
    )(x8, bwf, b.reshape(1, 1))


def kernel(leak_area, edge_index, pipe_attrs, params):
    src = edge_index[0]
    dst = edge_index[1]
    zeros_sb = jnp.zeros((SB, D), jnp.float32)
    ones_ch = jnp.ones((CH, D), jnp.float32)

    leak8 = leak_area.reshape(N8, 8)
    pa32 = pipe_attrs.reshape(E32, 128)

    x8 = _tc_node_embed(leak8, params['node_embed'][0], params['node_embed'][1])
    e32, q32 = _tc_edge_embed(pa32, params['edge_embed'][0],
                              params['edge_embed'][1], params['final_edge'][0],
                              params['final_edge'][1])
    e8 = e32.reshape(E8, 128)
    q = q32.reshape(E)

    cnt = _sc_count(dst, zeros_sb, ones_ch)
    c8 = cnt.reshape(NC, N8, 128)

    for lp in params['layers']:
        xi, xj = _sc_gather(x8.reshape(N, D), src, dst)
        msg8 = _tc_edge_mlp8(xi.reshape(E8, 128), xj.reshape(E8, 128), e8,
                             lp['e1'][0], lp['e1'][1], lp['e2'][0], lp['e2'][1])
        parts = _sc_scatter(msg8.reshape(E, D), dst, zeros_sb)
        p8 = parts.reshape(NC, N8, 128)
        x8 = _tc_node_mlp8(x8, p8[0], p8[1], c8[0], c8[1],
                           lp['n1'][0], lp['n1'][1], lp['n2'][0], lp['n2'][1])

    H = _tc_final(x8, params['final_node'][0], params['final_node'][1])
    return (H.reshape(N), q)
